# baseline (speedup 1.0000x reference)
import numpy as np
import jax
import jax.numpy as jnp
from jax import lax
from jax.experimental import pallas as pl
from jax.experimental.pallas import tpu as pltpu
from jax.experimental.pallas import tpu_sc as plsc

N_LEVELS = 16
F = 2
TABLE_SIZE = 2 ** 19
MASK = np.int32(TABLE_SIZE - 1)
_GROWTH = np.exp((np.log(4096.0) - np.log(16.0)) / (N_LEVELS - 1))
_SCALINGS = np.floor(16.0 * _GROWTH ** np.arange(N_LEVELS)).astype(np.float32)
P2 = np.int32(np.uint32(2654435761))
P3 = np.int32(805459861)

NC, NS = 2, 16
NW = NC * NS
B = 131072
BPW = B // NW
CH = 128
NCHUNK = BPW // CH
PV = CH // 16
NIDX = CH * N_LEVELS * 8 * F
OUT_CH = N_LEVELS * CH * F

_DN = lax.GatherDimensionNumbers(
    offset_dims=(), collapsed_slice_dims=(0,), start_index_map=(0,))


def _lane():
    return lax.iota(jnp.int32, 16)


def _dup(v, idx):
    return lax.gather(v, idx[:, None], _DN, (1,),
                      mode=lax.GatherScatterMode.PROMISE_IN_BOUNDS)


def _hash8(xi, yi, zi):
    px1 = xi + np.int32(1)
    py0 = yi * P2
    py1 = py0 + P2
    pz0 = zi * P3
    pz1 = pz0 + P3
    e11 = px1 ^ py1
    e10 = px1 ^ py0
    e00 = xi ^ py0
    e01 = xi ^ py1
    def h(e, pz):
        return (e ^ pz) & MASK
    return (h(e11, pz1), h(e10, pz1), h(e00, pz1), h(e01, pz1),
            h(e11, pz0), h(e10, pz0), h(e00, pz0), h(e01, pz0))


def _body(xh, yh, zh, table_hbm, out_hbm, x_v, y_v, z_v, idx_v, rows_v,
          out_v, sem):
    wid = lax.axis_index("s") * np.int32(NC) + lax.axis_index("c")
    base = wid * np.int32(BPW)

    for src, dst in ((xh, x_v), (yh, y_v), (zh, z_v)):
        pltpu.sync_copy(src.at[pl.ds(base, BPW)], dst)

    lane = _lane()
    dup_lo = lax.shift_right_logical(lane, np.int32(1))
    dup_hi = dup_lo + np.int32(8)
    par = lane & np.int32(1)

    @pl.loop(np.int32(0), np.int32(NCHUNK))
    def chunk_loop(g):
        cbase = g * np.int32(CH)

        @pl.loop(np.int32(0), np.int32(PV))
        def p_loop(p):
            off = cbase + p * np.int32(16)
            x = x_v[pl.ds(off, 16)]
            y = y_v[pl.ds(off, 16)]
            z = z_v[pl.ds(off, 16)]
            for l in range(N_LEVELS):
                s = _SCALINGS[l]
                xi = (x * s).astype(jnp.int32)
                yi = (y * s).astype(jnp.int32)
                zi = (z * s).astype(jnp.int32)
                hs = _hash8(xi, yi, zi)
                padd = par + np.int32(l * TABLE_SIZE * 2)
                q0 = (p * np.int32(N_LEVELS) + np.int32(l)) * np.int32(256)
                for c in range(8):
                    h2 = hs[c] << np.int32(1)
                    qc = q0 + np.int32(c * 32)
                    idx_v[pl.ds(qc, 16)] = _dup(h2, dup_lo) + padd
                    idx_v[pl.ds(qc + np.int32(16), 16)] = _dup(h2, dup_hi) + padd

        pltpu.async_copy(table_hbm.at[idx_v], rows_v, sem).wait()

        @pl.loop(np.int32(0), np.int32(PV))
        def p2_loop(p):
            off = cbase + p * np.int32(16)
            x = x_v[pl.ds(off, 16)]
            y = y_v[pl.ds(off, 16)]
            z = z_v[pl.ds(off, 16)]
            for l in range(N_LEVELS):
                s = _SCALINGS[l]
                xs = x * s
                ys = y * s
                zs = z * s
                ox = xs - xs.astype(jnp.int32).astype(jnp.float32)
                oy = ys - ys.astype(jnp.int32).astype(jnp.float32)
                oz = zs - zs.astype(jnp.int32).astype(jnp.float32)
                q0 = (p * np.int32(N_LEVELS) + np.int32(l)) * np.int32(256)
                for half, dup in ((0, dup_lo), (1, dup_hi)):
                    oxh = _dup(ox, dup)
                    oyh = _dup(oy, dup)
                    ozh = _dup(oz, dup)
                    mxh = np.float32(1.0) - oxh
                    myh = np.float32(1.0) - oyh
                    mzh = np.float32(1.0) - ozh
                    qh = q0 + np.int32(half * 16)
                    fv = [rows_v[pl.ds(qh + np.int32(c * 32), 16)]
                          for c in range(8)]
                    f03 = fv[0] * oxh + fv[3] * mxh
                    f12 = fv[1] * oxh + fv[2] * mxh
                    f56 = fv[5] * oxh + fv[6] * mxh
                    f47 = fv[4] * oxh + fv[7] * mxh
                    f0312 = f03 * oyh + f12 * myh
                    f4756 = f47 * oyh + f56 * myh
                    enc = f0312 * ozh + f4756 * mzh
                    opos = (np.int32(l * CH * F) + p * np.int32(32)
                            + np.int32(half * 16))
                    out_v[pl.ds(opos, 16)] = enc

        pltpu.sync_copy(
            out_v,
            out_hbm.at[pl.ds((wid * np.int32(NCHUNK) + g) * np.int32(OUT_CH),
                             OUT_CH)])


@jax.jit
def _encode(xs, ys, zs, table_flat):
    mesh = plsc.VectorSubcoreMesh(
        core_axis_name="c", subcore_axis_name="s",
        num_cores=NC, num_subcores=NS)
    k = pl.kernel(
        _body,
        out_type=jax.ShapeDtypeStruct((B * N_LEVELS * F,), jnp.float32),
        mesh=mesh,
        scratch_types=[
            pltpu.VMEM((BPW,), jnp.float32),
            pltpu.VMEM((BPW,), jnp.float32),
            pltpu.VMEM((BPW,), jnp.float32),
            pltpu.VMEM((NIDX,), jnp.int32),
            pltpu.VMEM((NIDX,), jnp.float32),
            pltpu.VMEM((OUT_CH,), jnp.float32),
            pltpu.SemaphoreType.DMA,
        ],
    )
    flat = k(xs, ys, zs, table_flat)
    arr = flat.reshape(NW, NCHUNK, N_LEVELS, CH, F)
    arr = arr.transpose(0, 1, 3, 2, 4)
    return arr.reshape(B, N_LEVELS * F)


def kernel(in_tensor, hash_table):
    pts = in_tensor.astype(jnp.float32)
    with jax.enable_x64(False):
        return _encode(pts[:, 0], pts[:, 1], pts[:, 2],
                       hash_table.astype(jnp.float32).reshape(-1))


# speedup vs baseline: 10.6392x; 10.6392x over previous
import numpy as np
import jax
import jax.numpy as jnp
from jax import lax
from jax.experimental import pallas as pl
from jax.experimental.pallas import tpu as pltpu
from jax.experimental.pallas import tpu_sc as plsc

N_LEVELS = 16
F = 2
TABLE_SIZE = 2 ** 19
MASK = np.int32(TABLE_SIZE - 1)
_GROWTH = np.exp((np.log(4096.0) - np.log(16.0)) / (N_LEVELS - 1))
_SCALINGS = np.floor(16.0 * _GROWTH ** np.arange(N_LEVELS)).astype(np.float32)
P2 = np.int32(np.uint32(2654435761))
P3 = np.int32(805459861)

NC, NS = 2, 16
NW = NC * NS
B = 131072
BPW = B // NW
CH = 64
NCHUNK = BPW // CH
PV = CH // 16
NIDX = CH * N_LEVELS * 8 * F
L2 = np.int32(2 * TABLE_SIZE)

_DN = lax.GatherDimensionNumbers(
    offset_dims=(), collapsed_slice_dims=(0,), start_index_map=(0,))


def _lane():
    return lax.iota(jnp.int32, 16)


def _dup(v, idx):
    return lax.gather(v, idx[:, None], _DN, (1,),
                      mode=lax.GatherScatterMode.PROMISE_IN_BOUNDS)


def _hash8(xi, yi, zi):
    px1 = xi + np.int32(1)
    py0 = yi * P2
    py1 = py0 + P2
    pz0 = zi * P3
    pz1 = pz0 + P3
    e11 = px1 ^ py1
    e10 = px1 ^ py0
    e00 = xi ^ py0
    e01 = xi ^ py1
    def h(e, pz):
        return (e ^ pz) & MASK
    return (h(e11, pz1), h(e10, pz1), h(e00, pz1), h(e01, pz1),
            h(e11, pz0), h(e10, pz0), h(e00, pz0), h(e01, pz0))


def _body(xh, yh, zh, table_hbm, scal_hbm, out_hbm,
          x_v, y_v, z_v, scal_v, idx_a, idx_b, rows_a, rows_b, out_v,
          sem_a, sem_b):
    wid = lax.axis_index("s") * np.int32(NC) + lax.axis_index("c")
    base = wid * np.int32(BPW)

    for src, dst in ((xh, x_v), (yh, y_v), (zh, z_v), (scal_hbm, scal_v)):
        pltpu.sync_copy(src.at[pl.ds(0, src.shape[0])]
                        if src is scal_hbm else src.at[pl.ds(base, BPW)], dst)

    lane = _lane()
    dup_lo = lax.shift_right_logical(lane, np.int32(1))
    dup_hi = dup_lo + np.int32(8)
    par128 = (lane & np.int32(1)) * np.int32(128)
    idx_even = (lane & np.int32(7)) << np.int32(1)
    idx_odd = idx_even + np.int32(1)
    is_lo = lane < np.int32(8)
    scal16 = scal_v[pl.ds(0, 16)]

    def phase1(idx_v, g):
        cbase = g * np.int32(CH)

        @pl.loop(np.int32(0), np.int32(N_LEVELS))
        def l_loop(l):
            s = _dup(scal16, jnp.full((16,), l, jnp.int32))
            ladd = par128 + l * np.int32(L2)

            @pl.loop(np.int32(0), np.int32(PV))
            def p_loop(p):
                off = cbase + p * np.int32(16)
                xi = (x_v[pl.ds(off, 16)] * s).astype(jnp.int32)
                yi = (y_v[pl.ds(off, 16)] * s).astype(jnp.int32)
                zi = (z_v[pl.ds(off, 16)] * s).astype(jnp.int32)
                hs = _hash8(xi, yi, zi)
                q0 = (p * np.int32(N_LEVELS) + l) * np.int32(256)
                for c in range(8):
                    qc = q0 + np.int32(c * 32)
                    tl = _dup(hs[c], dup_lo)
                    th = _dup(hs[c], dup_hi)
                    idx_v[pl.ds(qc, 16)] = tl + (tl & np.int32(-128)) + ladd
                    idx_v[pl.ds(qc + np.int32(16), 16)] = (
                        th + (th & np.int32(-128)) + ladd)

    def fire(idx_v, rows_v, sem):
        pltpu.async_copy(table_hbm.at[idx_v], rows_v, sem)

    def wait(idx_v, rows_v, sem):
        pltpu.make_async_copy(table_hbm.at[idx_v], rows_v, sem).wait()

    def phase2(rows_v, g, hi):
        cbase = g * np.int32(CH)

        @pl.loop(np.int32(0), np.int32(N_LEVELS))
        def l_loop(l):
            s = _dup(scal16, jnp.full((16,), l, jnp.int32))
            ch0 = l * np.int32(2)

            @pl.loop(np.int32(0), np.int32(PV))
            def p_loop(p):
                off = cbase + p * np.int32(16)
                xs = x_v[pl.ds(off, 16)] * s
                ys = y_v[pl.ds(off, 16)] * s
                zs = z_v[pl.ds(off, 16)] * s
                ox = xs - xs.astype(jnp.int32).astype(jnp.float32)
                oy = ys - ys.astype(jnp.int32).astype(jnp.float32)
                oz = zs - zs.astype(jnp.int32).astype(jnp.float32)
                q0 = (p * np.int32(N_LEVELS) + l) * np.int32(256)
                encs = []
                for dup in (dup_lo, dup_hi):
                    oxh = _dup(ox, dup)
                    oyh = _dup(oy, dup)
                    ozh = _dup(oz, dup)
                    mxh = np.float32(1.0) - oxh
                    myh = np.float32(1.0) - oyh
                    mzh = np.float32(1.0) - ozh
                    fv = [rows_v[pl.ds(q0 + np.int32(c * 32), 16)]
                          for c in range(8)] if dup is dup_lo else \
                         [rows_v[pl.ds(q0 + np.int32(c * 32 + 16), 16)]
                          for c in range(8)]
                    f03 = fv[0] * oxh + fv[3] * mxh
                    f12 = fv[1] * oxh + fv[2] * mxh
                    f56 = fv[5] * oxh + fv[6] * mxh
                    f47 = fv[4] * oxh + fv[7] * mxh
                    f0312 = f03 * oyh + f12 * myh
                    f4756 = f47 * oyh + f56 * myh
                    encs.append(f0312 * ozh + f4756 * mzh)
                vlo, vhi = encs
                for fi, didx in ((0, idx_even), (1, idx_odd)):
                    ch = ch0 + np.int32(fi)
                    merged = jnp.where(is_lo, _dup(vlo, didx),
                                       _dup(vhi, didx))
                    opos = ((ch >> np.int32(3)) * np.int32(1024)
                            + (ch & np.int32(7)) * np.int32(128)
                            + np.int32(hi * CH) + p * np.int32(16))
                    out_v[pl.ds(opos, 16)] = merged

    phase1(idx_a, np.int32(0))
    fire(idx_a, rows_a, sem_a)

    @pl.loop(np.int32(0), np.int32(NCHUNK), step=np.int32(2))
    def pair_loop(g):
        phase1(idx_b, g + np.int32(1))
        fire(idx_b, rows_b, sem_b)
        wait(idx_a, rows_a, sem_a)
        phase2(rows_a, g, 0)

        @pl.when(g + np.int32(2) < np.int32(NCHUNK))
        def _():
            phase1(idx_a, g + np.int32(2))
            fire(idx_a, rows_a, sem_a)

        wait(idx_b, rows_b, sem_b)
        phase2(rows_b, g + np.int32(1), 1)

        pbase = (wid * np.int32(NCHUNK // 2)
                 + lax.shift_right_logical(g, np.int32(1))) * np.int32(1024)
        for cb in range(4):
            pltpu.sync_copy(
                out_v.at[pl.ds(cb * 1024, 1024)],
                out_hbm.at[pl.ds(np.int32(cb * (2 ** 20)) + pbase, 1024)])


@jax.jit
def _encode(xs, ys, zs, table_flat, scalings):
    mesh = plsc.VectorSubcoreMesh(
        core_axis_name="c", subcore_axis_name="s",
        num_cores=NC, num_subcores=NS)
    k = pl.kernel(
        _body,
        out_type=jax.ShapeDtypeStruct((B * N_LEVELS * F,), jnp.float32),
        mesh=mesh,
        scratch_types=[
            pltpu.VMEM((BPW,), jnp.float32),
            pltpu.VMEM((BPW,), jnp.float32),
            pltpu.VMEM((BPW,), jnp.float32),
            pltpu.VMEM((16,), jnp.float32),
            pltpu.VMEM((NIDX,), jnp.int32),
            pltpu.VMEM((NIDX,), jnp.int32),
            pltpu.VMEM((NIDX,), jnp.float32),
            pltpu.VMEM((NIDX,), jnp.float32),
            pltpu.VMEM((N_LEVELS * F * 2 * CH,), jnp.float32),
            pltpu.SemaphoreType.DMA,
            pltpu.SemaphoreType.DMA,
        ],
    )
    flat = k(xs, ys, zs, table_flat, scalings)
    arr = flat.reshape(4, 1024, 8, 128)
    return arr.transpose(1, 3, 0, 2).reshape(B, N_LEVELS * F)


def kernel(in_tensor, hash_table):
    pts = in_tensor.astype(jnp.float32)
    with jax.enable_x64(False):
        tab = hash_table.astype(jnp.float32)
        tab = tab.reshape(65536, 128, F).transpose(0, 2, 1).reshape(-1)
        return _encode(pts[:, 0], pts[:, 1], pts[:, 2], tab,
                       jnp.asarray(_SCALINGS))
